# Initial kernel scaffold; baseline (speedup 1.0000x reference)
import jax
import jax.numpy as jnp
from jax.experimental import pallas as pl


def kernel(x, mask, pe):
    raise NotImplementedError("write your pallas kernel here")


# speedup vs baseline: 5.8561x; 5.8561x over previous
import math

import jax
import jax.numpy as jnp
from jax.experimental import pallas as pl


def _body(x_ref, m_ref, pe_ref, pad_ref, o_ref):
    x = x_ref[...]
    m = m_ref[...].astype(jnp.float32)
    pe = pe_ref[...]
    pad = pad_ref[...]
    scale = math.sqrt(x.shape[-1])
    add = pe[None, :, :] * (1.0 - m) + pad[None, :, :] * m
    o_ref[...] = x * scale + add


def kernel(x, mask, pe):
    B, L, D = x.shape
    pad_row = jax.lax.slice_in_dim(pe, pe.shape[0] - 1, pe.shape[0], axis=0)
    pe_l = jax.lax.slice_in_dim(pe, 0, L, axis=0)
    mask3 = mask.reshape(B, L, 1)
    bB = 32
    grid = (B // bB,)
    return pl.pallas_call(
        _body,
        grid=grid,
        in_specs=[
            pl.BlockSpec((bB, L, D), lambda i: (i, 0, 0)),
            pl.BlockSpec((bB, L, 1), lambda i: (i, 0, 0)),
            pl.BlockSpec((L, D), lambda i: (0, 0)),
            pl.BlockSpec((1, D), lambda i: (0, 0)),
        ],
        out_specs=pl.BlockSpec((bB, L, D), lambda i: (i, 0, 0)),
        out_shape=jax.ShapeDtypeStruct((B, L, D), x.dtype),
    )(x, mask3, pe_l, pad_row)


# speedup vs baseline: 6.1036x; 1.0423x over previous
import math

import jax
import jax.numpy as jnp
from jax.experimental import pallas as pl


def _body(x_ref, m_ref, pe_ref, diff_ref, o_ref):
    x = x_ref[...]
    m = m_ref[...]
    pe = pe_ref[...]
    diff = diff_ref[...]
    scale = math.sqrt(x.shape[-1])
    o_ref[...] = x * scale + (pe[None, :, :] - m * diff[None, :, :])


def kernel(x, mask, pe):
    B, L, D = x.shape
    pad_row = jax.lax.slice_in_dim(pe, pe.shape[0] - 1, pe.shape[0], axis=0)
    pe_l = jax.lax.slice_in_dim(pe, 0, L, axis=0)
    diff = pe_l - pad_row
    mask3 = mask.astype(x.dtype).reshape(B, L, 1)
    bB = 64
    grid = (B // bB,)
    return pl.pallas_call(
        _body,
        grid=grid,
        in_specs=[
            pl.BlockSpec((bB, L, D), lambda i: (i, 0, 0)),
            pl.BlockSpec((bB, L, 1), lambda i: (i, 0, 0)),
            pl.BlockSpec((L, D), lambda i: (0, 0)),
            pl.BlockSpec((L, D), lambda i: (0, 0)),
        ],
        out_specs=pl.BlockSpec((bB, L, D), lambda i: (i, 0, 0)),
        out_shape=jax.ShapeDtypeStruct((B, L, D), x.dtype),
    )(x, mask3, pe_l, diff)


# speedup vs baseline: 12.7821x; 2.0942x over previous
import math

import jax
import jax.numpy as jnp
from jax.experimental import pallas as pl


def _body(x_ref, m_ref, pe_ref, diff_ref, o_ref):
    x = x_ref[...]
    m = m_ref[...]
    pe = pe_ref[...]
    diff = diff_ref[...]
    scale = math.sqrt(x.shape[-1])
    m3 = jax.lax.broadcast_in_dim(m, x.shape, (0, 1))
    o_ref[...] = x * scale + (pe[None, :, :] - m3 * diff[None, :, :])


def kernel(x, mask, pe):
    B, L, D = x.shape
    pad_row = jax.lax.slice_in_dim(pe, pe.shape[0] - 1, pe.shape[0], axis=0)
    pe_l = jax.lax.slice_in_dim(pe, 0, L, axis=0)
    diff = pe_l - pad_row
    mask_f = mask.astype(x.dtype)
    bB = 64
    grid = (B // bB,)
    return pl.pallas_call(
        _body,
        grid=grid,
        in_specs=[
            pl.BlockSpec((bB, L, D), lambda i: (i, 0, 0)),
            pl.BlockSpec((bB, L), lambda i: (i, 0)),
            pl.BlockSpec((L, D), lambda i: (0, 0)),
            pl.BlockSpec((L, D), lambda i: (0, 0)),
        ],
        out_specs=pl.BlockSpec((bB, L, D), lambda i: (i, 0, 0)),
        out_shape=jax.ShapeDtypeStruct((B, L, D), x.dtype),
    )(x, mask_f, pe_l, diff)


# speedup vs baseline: 12.8240x; 1.0033x over previous
import math

import jax
import jax.numpy as jnp
from jax.experimental import pallas as pl


def _body(x_ref, m_ref, pe_ref, diff_ref, o_ref):
    x = x_ref[...]
    m = m_ref[...]
    pe = pe_ref[...]
    diff = diff_ref[...]
    scale = math.sqrt(x.shape[-1])
    m3 = jax.lax.broadcast_in_dim(m, x.shape, (0, 1))
    o_ref[...] = x * scale + (pe[None, :, :] - m3 * diff[None, :, :])


def kernel(x, mask, pe):
    B, L, D = x.shape
    pad_row = jax.lax.slice_in_dim(pe, pe.shape[0] - 1, pe.shape[0], axis=0)
    pe_l = jax.lax.slice_in_dim(pe, 0, L, axis=0)
    diff = pe_l - pad_row
    mask_f = mask.astype(x.dtype)
    bB = 128
    grid = (B // bB,)
    return pl.pallas_call(
        _body,
        grid=grid,
        in_specs=[
            pl.BlockSpec((bB, L, D), lambda i: (i, 0, 0)),
            pl.BlockSpec((bB, L), lambda i: (i, 0)),
            pl.BlockSpec((L, D), lambda i: (0, 0)),
            pl.BlockSpec((L, D), lambda i: (0, 0)),
        ],
        out_specs=pl.BlockSpec((bB, L, D), lambda i: (i, 0, 0)),
        out_shape=jax.ShapeDtypeStruct((B, L, D), x.dtype),
    )(x, mask_f, pe_l, diff)
